# baseline (speedup 1.0000x reference)
import functools

import jax
import jax.numpy as jnp
from jax.experimental import pallas as pl
from jax.experimental.pallas import tpu as pltpu

_PREC = jax.lax.Precision.DEFAULT
_INV_SQRT2 = 0.7071067811865476
_NEG = float("-inf")


def _mm(a, b):
    return jax.lax.dot_general(
        a, b, (((1,), (0,)), ((), ())),
        precision=_PREC, preferred_element_type=jnp.float32)


def _mmT(a, b):
    return jax.lax.dot_general(
        a, b, (((1,), (1,)), ((), ())),
        precision=_PREC, preferred_element_type=jnp.float32)


def _pb_body(fpw_ref, w_ref, out_ref):
    j = pl.program_id(0)

    @pl.when(j == 0)
    def _():
        out_ref[...] = jnp.zeros_like(out_ref)

    out_ref[...] += _mm(fpw_ref[...], w_ref[...])


def _main_body(hs_ref, fk_ref, pb_ref, w1_ref, b1_ref, w2_ref, b2_ref,
               probs_ref, resid_ref, sp_ref, idx_ref, vals_ref, sl_ref,
               *, top_k):
    tblk = hs_ref.shape[0]
    dsz = w2_ref.shape[0]

    hs = hs_ref[...]
    scores = _mmT(hs, fk_ref[...])
    mx = jnp.max(scores, axis=-1, keepdims=True)
    e = jnp.exp(scores - mx)
    probs = e / jnp.sum(e, axis=-1, keepdims=True)
    probs_ref[...] = probs
    resid = hs - _mm(probs, pb_ref[...])
    resid_ref[...] = resid
    pre = _mmT(resid, w1_ref[...]) + b1_ref[...]
    h = 0.5 * pre * (1.0 + jax.lax.erf(pre * _INV_SQRT2))
    c = _mmT(h, w2_ref[...]) + b2_ref[...]

    lane = jax.lax.broadcasted_iota(jnp.int32, (tblk, dsz), 1)
    lane8 = jax.lax.broadcasted_iota(jnp.int32, (tblk, top_k), 1)
    work = c
    idx8 = jnp.zeros((tblk, top_k), jnp.int32)
    val8 = jnp.zeros((tblk, top_k), jnp.float32)
    for k in range(top_k):
        mxv = jnp.max(work, axis=-1, keepdims=True)
        ii = jnp.min(jnp.where(work == mxv, lane, dsz), axis=-1,
                     keepdims=True)
        work = jnp.where(lane == ii, _NEG, work)
        idx8 = jnp.where(lane8 == k, ii, idx8)
        val8 = jnp.where(lane8 == k, mxv, val8)
    sp_ref[...] = jnp.where(work == _NEG, c, 0.0)
    idx_ref[...] = idx8
    vals_ref[...] = val8
    sl_ref[...] = jnp.sum(jnp.abs(val8)).reshape(1, 1, 1)


def _loss_body(sp_ref, w_ref, resid_ref, out_ref):
    dd = _mm(sp_ref[...], w_ref[...]) - resid_ref[...]
    out_ref[...] = jnp.sum(dd * dd).reshape(1, 1, 1)


def kernel(hidden_states, W, family_keys, family_proj_w, W1, b1, W2, b2):
    B, T, D = hidden_states.shape
    dict_size, _ = W.shape
    nf = family_keys.shape[0]
    top_k = 8

    x = hidden_states.reshape(T, D)
    b1r = b1.reshape(1, D)
    b2r = b2.reshape(1, dict_size)

    npb = 4
    pbk = dict_size // npb
    pb = pl.pallas_call(
        _pb_body,
        grid=(npb,),
        in_specs=[
            pl.BlockSpec((nf, pbk), lambda j: (0, j)),
            pl.BlockSpec((pbk, D), lambda j: (j, 0)),
        ],
        out_specs=pl.BlockSpec((nf, D), lambda j: (0, 0)),
        out_shape=jax.ShapeDtypeStruct((nf, D), jnp.float32),
    )(family_proj_w, W)

    tblk = 64 if T % 64 == 0 else T
    nt = T // tblk

    body = functools.partial(_main_body, top_k=top_k)
    probs, resid, sp, idx, vals, slp = pl.pallas_call(
        body,
        grid=(nt,),
        in_specs=[
            pl.BlockSpec((tblk, D), lambda i: (i, 0)),
            pl.BlockSpec((nf, D), lambda i: (0, 0)),
            pl.BlockSpec((nf, D), lambda i: (0, 0)),
            pl.BlockSpec((D, D), lambda i: (0, 0)),
            pl.BlockSpec((1, D), lambda i: (0, 0)),
            pl.BlockSpec((dict_size, D), lambda i: (0, 0)),
            pl.BlockSpec((1, dict_size), lambda i: (0, 0)),
        ],
        out_specs=[
            pl.BlockSpec((tblk, nf), lambda i: (i, 0)),
            pl.BlockSpec((tblk, D), lambda i: (i, 0)),
            pl.BlockSpec((tblk, dict_size), lambda i: (i, 0)),
            pl.BlockSpec((tblk, top_k), lambda i: (i, 0)),
            pl.BlockSpec((tblk, top_k), lambda i: (i, 0)),
            pl.BlockSpec((1, 1, 1), lambda i: (i, 0, 0)),
        ],
        out_shape=[
            jax.ShapeDtypeStruct((T, nf), jnp.float32),
            jax.ShapeDtypeStruct((T, D), jnp.float32),
            jax.ShapeDtypeStruct((T, dict_size), jnp.float32),
            jax.ShapeDtypeStruct((T, top_k), jnp.int32),
            jax.ShapeDtypeStruct((T, top_k), jnp.float32),
            jax.ShapeDtypeStruct((nt, 1, 1), jnp.float32),
        ],
        compiler_params=pltpu.CompilerParams(
            dimension_semantics=("arbitrary",)),
    )(x, family_keys, pb, W1, b1r, W2, b2r)

    lossp = pl.pallas_call(
        _loss_body,
        grid=(nt,),
        in_specs=[
            pl.BlockSpec((tblk, dict_size), lambda i: (i, 0)),
            pl.BlockSpec((dict_size, D), lambda i: (0, 0)),
            pl.BlockSpec((tblk, D), lambda i: (i, 0)),
        ],
        out_specs=pl.BlockSpec((1, 1, 1), lambda i: (i, 0, 0)),
        out_shape=jax.ShapeDtypeStruct((nt, 1, 1), jnp.float32),
    )(sp, W, resid)

    recon_loss = jnp.sum(lossp) / (T * D)
    sparsity_loss = jnp.sum(slp) / (T * dict_size)
    return (recon_loss, sparsity_loss,
            sp.reshape(B, T, dict_size), probs.reshape(B, T, nf))


# speedup vs baseline: 2.1909x; 2.1909x over previous
import functools

import jax
import jax.numpy as jnp
from jax.experimental import pallas as pl
from jax.experimental.pallas import tpu as pltpu

_PREC = jax.lax.Precision.DEFAULT
_INV_SQRT2 = 0.7071067811865476
_NEG = float("-inf")


def _mm(a, b):
    return jax.lax.dot_general(
        a, b, (((1,), (0,)), ((), ())),
        precision=_PREC, preferred_element_type=jnp.float32)


def _mmT(a, b):
    return jax.lax.dot_general(
        a, b, (((1,), (1,)), ((), ())),
        precision=_PREC, preferred_element_type=jnp.float32)


def _pb_body(fpw_ref, w_ref, out_ref):
    j = pl.program_id(0)

    @pl.when(j == 0)
    def _():
        out_ref[...] = jnp.zeros_like(out_ref)

    out_ref[...] += _mm(fpw_ref[...], w_ref[...])


def _stage1_body(hs_ref, fk_ref, pb_ref, w1_ref, b1_ref,
                 probs_ref, resid_ref, h_ref):
    hs = hs_ref[...]
    scores = _mmT(hs, fk_ref[...])
    mx = jnp.max(scores, axis=-1, keepdims=True)
    e = jnp.exp(scores - mx)
    probs = e / jnp.sum(e, axis=-1, keepdims=True)
    probs_ref[...] = probs
    resid = hs - _mm(probs, pb_ref[...])
    resid_ref[...] = resid
    pre = _mmT(resid, w1_ref[...]) + b1_ref[...]
    h_ref[...] = 0.5 * pre * (1.0 + jax.lax.erf(pre * _INV_SQRT2))


def _topk_body(h_ref, w2_ref, b2_ref,
               sp_ref, idx_ref, vals_ref, sl_ref, *, top_k):
    tblk = h_ref.shape[0]
    dsz = w2_ref.shape[0]
    c = _mmT(h_ref[...], w2_ref[...]) + b2_ref[...]

    lane = jax.lax.broadcasted_iota(jnp.int32, (tblk, dsz), 1)
    lane8 = jax.lax.broadcasted_iota(jnp.int32, (tblk, top_k), 1)
    work = c
    idx8 = jnp.zeros((tblk, top_k), jnp.int32)
    val8 = jnp.zeros((tblk, top_k), jnp.float32)
    for k in range(top_k):
        mxv = jnp.max(work, axis=-1, keepdims=True)
        ii = jnp.min(jnp.where(work == mxv, lane, dsz), axis=-1,
                     keepdims=True)
        work = jnp.where(lane == ii, _NEG, work)
        idx8 = jnp.where(lane8 == k, ii, idx8)
        val8 = jnp.where(lane8 == k, mxv, val8)
    sp_ref[...] = jnp.where(work == _NEG, c, 0.0)
    idx_ref[...] = idx8
    vals_ref[...] = val8
    sl_ref[...] = jnp.sum(jnp.abs(val8)).reshape(1, 1, 1)


def _loss_body(sp_ref, w_ref, resid_ref, out_ref):
    dd = _mm(sp_ref[...], w_ref[...]) - resid_ref[...]
    out_ref[...] = jnp.sum(dd * dd).reshape(1, 1, 1)


def kernel(hidden_states, W, family_keys, family_proj_w, W1, b1, W2, b2):
    B, T, D = hidden_states.shape
    dict_size, _ = W.shape
    nf = family_keys.shape[0]
    top_k = 8

    x = hidden_states.reshape(T, D)
    b1r = b1.reshape(1, D)
    b2r = b2.reshape(1, dict_size)

    npb = 4
    pbk = dict_size // npb
    pb = pl.pallas_call(
        _pb_body,
        grid=(npb,),
        in_specs=[
            pl.BlockSpec((nf, pbk), lambda j: (0, j)),
            pl.BlockSpec((pbk, D), lambda j: (j, 0)),
        ],
        out_specs=pl.BlockSpec((nf, D), lambda j: (0, 0)),
        out_shape=jax.ShapeDtypeStruct((nf, D), jnp.float32),
    )(family_proj_w, W)

    tblk = 256 if T % 256 == 0 else T
    nt = T // tblk

    probs, resid, h = pl.pallas_call(
        _stage1_body,
        grid=(nt,),
        in_specs=[
            pl.BlockSpec((tblk, D), lambda i: (i, 0)),
            pl.BlockSpec((nf, D), lambda i: (0, 0)),
            pl.BlockSpec((nf, D), lambda i: (0, 0)),
            pl.BlockSpec((D, D), lambda i: (0, 0)),
            pl.BlockSpec((1, D), lambda i: (0, 0)),
        ],
        out_specs=[
            pl.BlockSpec((tblk, nf), lambda i: (i, 0)),
            pl.BlockSpec((tblk, D), lambda i: (i, 0)),
            pl.BlockSpec((tblk, D), lambda i: (i, 0)),
        ],
        out_shape=[
            jax.ShapeDtypeStruct((T, nf), jnp.float32),
            jax.ShapeDtypeStruct((T, D), jnp.float32),
            jax.ShapeDtypeStruct((T, D), jnp.float32),
        ],
        compiler_params=pltpu.CompilerParams(
            dimension_semantics=("arbitrary",)),
    )(x, family_keys, pb, W1, b1r)

    body = functools.partial(_topk_body, top_k=top_k)
    sp, idx, vals, slp = pl.pallas_call(
        body,
        grid=(nt,),
        in_specs=[
            pl.BlockSpec((tblk, D), lambda i: (i, 0)),
            pl.BlockSpec((dict_size, D), lambda i: (0, 0)),
            pl.BlockSpec((1, dict_size), lambda i: (0, 0)),
        ],
        out_specs=[
            pl.BlockSpec((tblk, dict_size), lambda i: (i, 0)),
            pl.BlockSpec((tblk, top_k), lambda i: (i, 0)),
            pl.BlockSpec((tblk, top_k), lambda i: (i, 0)),
            pl.BlockSpec((1, 1, 1), lambda i: (i, 0, 0)),
        ],
        out_shape=[
            jax.ShapeDtypeStruct((T, dict_size), jnp.float32),
            jax.ShapeDtypeStruct((T, top_k), jnp.int32),
            jax.ShapeDtypeStruct((T, top_k), jnp.float32),
            jax.ShapeDtypeStruct((nt, 1, 1), jnp.float32),
        ],
        compiler_params=pltpu.CompilerParams(
            dimension_semantics=("arbitrary",)),
    )(h, W2, b2r)

    lossp = pl.pallas_call(
        _loss_body,
        grid=(nt,),
        in_specs=[
            pl.BlockSpec((tblk, dict_size), lambda i: (i, 0)),
            pl.BlockSpec((dict_size, D), lambda i: (0, 0)),
            pl.BlockSpec((tblk, D), lambda i: (i, 0)),
        ],
        out_specs=pl.BlockSpec((1, 1, 1), lambda i: (i, 0, 0)),
        out_shape=jax.ShapeDtypeStruct((nt, 1, 1), jnp.float32),
    )(sp, W, resid)

    recon_loss = jnp.sum(lossp) / (T * D)
    sparsity_loss = jnp.sum(slp) / (T * dict_size)
    return (recon_loss, sparsity_loss,
            sp.reshape(B, T, dict_size), probs.reshape(B, T, nf))
